# baseline (speedup 1.0000x reference)
import functools

import jax
import jax.numpy as jnp
from jax.experimental import pallas as pl
from jax.experimental.pallas import tpu as pltpu

_N_BINS = 10
_LANES = 128


def _ece_kernel(lo_ref, up_ref, x_ref, lab_ref,
                ece_ref, accb_ref, oe_ref, prop_ref, ce_ref,
                cnt_ref, sc_ref, sa_ref,
                *, n_total, n_cols, n_steps):
    i = pl.program_id(0)

    @pl.when(i == 0)
    def _init():
        z = jnp.zeros((1, _LANES), jnp.float32)
        cnt_ref[...] = z
        sc_ref[...] = z
        sa_ref[...] = z

    x = x_ref[...]
    lab = lab_ref[...]
    m = jnp.max(x, axis=1, keepdims=True)
    e = jnp.exp(x - m)
    s = jnp.sum(e, axis=1, keepdims=True)
    conf = 1.0 / s
    col = jax.lax.broadcasted_iota(jnp.int32, x.shape, 1)
    pidx = jnp.min(jnp.where(x == m, col, n_cols), axis=1, keepdims=True)
    acc = (pidx == lab).astype(jnp.float32)

    lo = lo_ref[...]
    up = up_ref[...]
    in_bin = ((conf > lo) & (conf <= up)).astype(jnp.float32)
    cnt_ref[...] += jnp.sum(in_bin, axis=0, keepdims=True)
    sc_ref[...] += jnp.sum(in_bin * conf, axis=0, keepdims=True)
    sa_ref[...] += jnp.sum(in_bin * acc, axis=0, keepdims=True)

    @pl.when(i == n_steps - 1)
    def _fin():
        cnt = cnt_ref[...]
        nf = jnp.float32(n_total)
        prop = cnt / nf
        denom = jnp.maximum(cnt, 1.0)
        accb = sa_ref[...] / denom
        avgc = sc_ref[...] / denom
        ce = avgc - accb
        ece_ref[...] = jnp.sum(jnp.abs(ce) * prop, keepdims=True)
        oe_ref[...] = jnp.sum(avgc * jnp.maximum(ce, 0.0) * prop, keepdims=True)
        accb_ref[...] = accb
        prop_ref[...] = prop
        ce_ref[...] = jnp.abs(ce)


def _pick_block(n):
    for b in (8000, 4000, 2000, 1000, 800, 400, 200, 100, 40, 8, 4, 2, 1):
        if n % b == 0:
            return b
    return n


def kernel(logits, labels):
    n, c = logits.shape
    blk = _pick_block(n)
    steps = n // blk
    bounds = jnp.linspace(0.0, 1.0, _N_BINS + 1)
    lo = jnp.full((1, _LANES), jnp.inf, jnp.float32).at[0, :_N_BINS].set(bounds[:-1])
    up = jnp.full((1, _LANES), -jnp.inf, jnp.float32).at[0, :_N_BINS].set(bounds[1:])
    lab2 = labels.reshape(n, 1)

    small = jax.ShapeDtypeStruct((1, 1), jnp.float32)
    wide = jax.ShapeDtypeStruct((1, _LANES), jnp.float32)
    const_spec = pl.BlockSpec((1, _LANES), lambda i: (0, 0))
    scalar_spec = pl.BlockSpec((1, 1), lambda i: (0, 0))
    outs = pl.pallas_call(
        functools.partial(_ece_kernel, n_total=n, n_cols=c, n_steps=steps),
        grid=(steps,),
        in_specs=[
            const_spec,
            const_spec,
            pl.BlockSpec((blk, c), lambda i: (i, 0)),
            pl.BlockSpec((blk, 1), lambda i: (i, 0)),
        ],
        out_specs=[scalar_spec, const_spec, scalar_spec, const_spec, const_spec],
        out_shape=[small, wide, small, wide, wide],
        scratch_shapes=[pltpu.VMEM((1, _LANES), jnp.float32)] * 3,
        compiler_params=pltpu.CompilerParams(dimension_semantics=("arbitrary",)),
    )(lo, up, logits, lab2)
    ece, accb, oe, prop, ce = outs
    return (ece.reshape(()), accb[0, :_N_BINS], oe.reshape(()),
            prop[0, :_N_BINS], ce[0, :_N_BINS])


# speedup vs baseline: 1.8152x; 1.8152x over previous
import functools

import jax
import jax.numpy as jnp
from jax.experimental import pallas as pl
from jax.experimental.pallas import tpu as pltpu

_N_BINS = 10
_LANES = 128
_GROUP = 128
_SUPER = 8
_BLK = 2048


def _ece_kernel(bounds_ref, x_ref, lab_ref,
                ece_ref, accb_ref, oe_ref, prop_ref, ce_ref,
                cnt_ref, sc_ref, sa_ref,
                *, n_total, n_cols, n_steps):
    i = pl.program_id(0)

    @pl.when(i == 0)
    def _init():
        z = jnp.zeros((_N_BINS, 8, _LANES), jnp.float32)
        cnt_ref[...] = z
        sc_ref[...] = z
        sa_ref[...] = z

    ci = jax.lax.broadcasted_iota(jnp.int32, (n_cols, _LANES), 0)
    ridx = (jax.lax.broadcasted_iota(jnp.int32, (_SUPER, _LANES), 0) * _GROUP
            + jax.lax.broadcasted_iota(jnp.int32, (_SUPER, _LANES), 1))

    for sg in range(_BLK // (_GROUP * _SUPER)):
        confs = []
        accs = []
        for g in range(_SUPER):
            r0 = (sg * _SUPER + g) * _GROUP
            xg = x_ref[r0:r0 + _GROUP, :]
            xt = xg.T
            e = jnp.exp(xt)
            mx = jnp.max(e, axis=0, keepdims=True)
            s = jnp.sum(e, axis=0, keepdims=True)
            pidx = jnp.min(jnp.where(e == mx, ci, n_cols),
                           axis=0, keepdims=True)
            labg = lab_ref[0:1, r0:r0 + _GROUP]
            confs.append(mx / s)
            accs.append(jnp.where(pidx == labg, 1.0, 0.0))
        conf8 = jnp.concatenate(confs, axis=0)
        acc8 = jnp.concatenate(accs, axis=0)
        base = i * _BLK + sg * _GROUP * _SUPER
        valid = (base + ridx) < n_total
        conf8 = jnp.where(valid, conf8, 0.0)
        for b in range(_N_BINS):
            cond = (conf8 > bounds_ref[b]) & (conf8 <= bounds_ref[b + 1])
            cnt_ref[b] += jnp.where(cond, 1.0, 0.0)
            sc_ref[b] += jnp.where(cond, conf8, 0.0)
            sa_ref[b] += jnp.where(cond, acc8, 0.0)

    @pl.when(i == n_steps - 1)
    def _fin():
        nf = jnp.float32(n_total)
        ece = jnp.zeros((1, 1), jnp.float32)
        oe = jnp.zeros((1, 1), jnp.float32)
        for b in range(_N_BINS):
            cnt = jnp.sum(cnt_ref[b], keepdims=True)[:1, :1]
            sc = jnp.sum(sc_ref[b], keepdims=True)[:1, :1]
            sa = jnp.sum(sa_ref[b], keepdims=True)[:1, :1]
            prop = cnt / nf
            denom = jnp.maximum(cnt, 1.0)
            accb = sa / denom
            avgc = sc / denom
            ce = avgc - accb
            ece = ece + jnp.abs(ce) * prop
            oe = oe + avgc * jnp.maximum(ce, 0.0) * prop
            accb_ref[pl.ds(b, 1), :] = accb
            prop_ref[pl.ds(b, 1), :] = prop
            ce_ref[pl.ds(b, 1), :] = jnp.abs(ce)
        ece_ref[...] = ece
        oe_ref[...] = oe


def kernel(logits, labels):
    n, c = logits.shape
    steps = pl.cdiv(n, _BLK)
    bounds = jnp.linspace(0.0, 1.0, _N_BINS + 1)
    lab2 = labels.reshape(1, n)

    outs = pl.pallas_call(
        functools.partial(_ece_kernel, n_total=n, n_cols=c, n_steps=steps),
        grid=(steps,),
        in_specs=[
            pl.BlockSpec(memory_space=pltpu.SMEM),
            pl.BlockSpec((_BLK, c), lambda i: (i, 0)),
            pl.BlockSpec((1, _BLK), lambda i: (0, i)),
        ],
        out_specs=[
            pl.BlockSpec((1, 1), lambda i: (0, 0)),
            pl.BlockSpec((_N_BINS, 1), lambda i: (0, 0)),
            pl.BlockSpec((1, 1), lambda i: (0, 0)),
            pl.BlockSpec((_N_BINS, 1), lambda i: (0, 0)),
            pl.BlockSpec((_N_BINS, 1), lambda i: (0, 0)),
        ],
        out_shape=[
            jax.ShapeDtypeStruct((1, 1), jnp.float32),
            jax.ShapeDtypeStruct((_N_BINS, 1), jnp.float32),
            jax.ShapeDtypeStruct((1, 1), jnp.float32),
            jax.ShapeDtypeStruct((_N_BINS, 1), jnp.float32),
            jax.ShapeDtypeStruct((_N_BINS, 1), jnp.float32),
        ],
        scratch_shapes=[pltpu.VMEM((_N_BINS, 8, _LANES), jnp.float32)] * 3,
        compiler_params=pltpu.CompilerParams(dimension_semantics=("arbitrary",)),
    )(bounds, logits, lab2)
    ece, accb, oe, prop, ce = outs
    return (ece.reshape(()), accb[:, 0], oe.reshape(()), prop[:, 0], ce[:, 0])


# speedup vs baseline: 2.3764x; 1.3092x over previous
import functools

import jax
import jax.numpy as jnp
from jax.experimental import pallas as pl
from jax.experimental.pallas import tpu as pltpu

_N_BINS = 10
_LANES = 128
_GROUP = 128
_SUPER = 8
_BLK = 8192


def _ece_kernel(bounds_ref, x_ref, lab_ref,
                ece_ref, accb_ref, oe_ref, prop_ref, ce_ref,
                cnt_ref, sc_ref, sa_ref,
                *, n_total, n_cols, n_steps):
    i = pl.program_id(0)

    @pl.when(i == 0)
    def _init():
        z = jnp.zeros((_N_BINS, 8, _LANES), jnp.float32)
        cnt_ref[...] = z
        sc_ref[...] = z
        sa_ref[...] = z

    ci = jax.lax.broadcasted_iota(jnp.int32, (n_cols, _LANES), 0)
    ridx = (jax.lax.broadcasted_iota(jnp.int32, (_SUPER, _LANES), 0) * _GROUP
            + jax.lax.broadcasted_iota(jnp.int32, (_SUPER, _LANES), 1))

    for sg in range(_BLK // (_GROUP * _SUPER)):
        confs = []
        accs = []
        for g in range(_SUPER):
            r0 = (sg * _SUPER + g) * _GROUP
            xg = x_ref[r0:r0 + _GROUP, :]
            xt = xg.T
            e = jnp.exp(xt)
            mx = jnp.max(e, axis=0, keepdims=True)
            s = jnp.sum(e, axis=0, keepdims=True)
            pidx = jnp.min(jnp.where(e == mx, ci, n_cols),
                           axis=0, keepdims=True)
            labg = lab_ref[0:1, r0:r0 + _GROUP]
            confs.append(mx / s)
            accs.append(jnp.where(pidx == labg, 1.0, 0.0))
        conf8 = jnp.concatenate(confs, axis=0)
        acc8 = jnp.concatenate(accs, axis=0)
        base = i * _BLK + sg * _GROUP * _SUPER
        valid = (base + ridx) < n_total
        conf8 = jnp.where(valid, conf8, 0.0)
        for b in range(_N_BINS):
            cond = (conf8 > bounds_ref[b]) & (conf8 <= bounds_ref[b + 1])
            cnt_ref[b] += jnp.where(cond, 1.0, 0.0)
            sc_ref[b] += jnp.where(cond, conf8, 0.0)
            sa_ref[b] += jnp.where(cond, acc8, 0.0)

    @pl.when(i == n_steps - 1)
    def _fin():
        nf = jnp.float32(n_total)
        ece = jnp.zeros((1, 1), jnp.float32)
        oe = jnp.zeros((1, 1), jnp.float32)
        for b in range(_N_BINS):
            cnt = jnp.sum(cnt_ref[b], keepdims=True)[:1, :1]
            sc = jnp.sum(sc_ref[b], keepdims=True)[:1, :1]
            sa = jnp.sum(sa_ref[b], keepdims=True)[:1, :1]
            prop = cnt / nf
            denom = jnp.maximum(cnt, 1.0)
            accb = sa / denom
            avgc = sc / denom
            ce = avgc - accb
            ece = ece + jnp.abs(ce) * prop
            oe = oe + avgc * jnp.maximum(ce, 0.0) * prop
            accb_ref[pl.ds(b, 1), :] = accb
            prop_ref[pl.ds(b, 1), :] = prop
            ce_ref[pl.ds(b, 1), :] = jnp.abs(ce)
        ece_ref[...] = ece
        oe_ref[...] = oe


def kernel(logits, labels):
    n, c = logits.shape
    steps = pl.cdiv(n, _BLK)
    bounds = jnp.linspace(0.0, 1.0, _N_BINS + 1)
    lab2 = labels.reshape(1, n)

    outs = pl.pallas_call(
        functools.partial(_ece_kernel, n_total=n, n_cols=c, n_steps=steps),
        grid=(steps,),
        in_specs=[
            pl.BlockSpec(memory_space=pltpu.SMEM),
            pl.BlockSpec((_BLK, c), lambda i: (i, 0)),
            pl.BlockSpec((1, _BLK), lambda i: (0, i)),
        ],
        out_specs=[
            pl.BlockSpec((1, 1), lambda i: (0, 0)),
            pl.BlockSpec((_N_BINS, 1), lambda i: (0, 0)),
            pl.BlockSpec((1, 1), lambda i: (0, 0)),
            pl.BlockSpec((_N_BINS, 1), lambda i: (0, 0)),
            pl.BlockSpec((_N_BINS, 1), lambda i: (0, 0)),
        ],
        out_shape=[
            jax.ShapeDtypeStruct((1, 1), jnp.float32),
            jax.ShapeDtypeStruct((_N_BINS, 1), jnp.float32),
            jax.ShapeDtypeStruct((1, 1), jnp.float32),
            jax.ShapeDtypeStruct((_N_BINS, 1), jnp.float32),
            jax.ShapeDtypeStruct((_N_BINS, 1), jnp.float32),
        ],
        scratch_shapes=[pltpu.VMEM((_N_BINS, 8, _LANES), jnp.float32)] * 3,
        compiler_params=pltpu.CompilerParams(dimension_semantics=("arbitrary",)),
    )(bounds, logits, lab2)
    ece, accb, oe, prop, ce = outs
    return (ece.reshape(()), accb[:, 0], oe.reshape(()), prop[:, 0], ce[:, 0])


# speedup vs baseline: 8.1438x; 3.4270x over previous
import functools

import jax
import jax.numpy as jnp
from jax.experimental import pallas as pl
from jax.experimental.pallas import tpu as pltpu

_N_BINS = 10
_LANES = 128
_GROUP = 128
_SUPER = 8
_BLK = 16384


def _ece_kernel(bounds_ref, x_ref, lab_ref,
                ece_ref, accb_ref, oe_ref, prop_ref, ce_ref,
                cnt_ref, sc_ref, sa_ref,
                *, n_total, n_cols, n_steps):
    i = pl.program_id(0)

    @pl.when(i == 0)
    def _init():
        z = jnp.zeros((_N_BINS, 8, _LANES), jnp.float32)
        cnt_ref[...] = z
        sc_ref[...] = z
        sa_ref[...] = z

    sub8 = jax.lax.broadcasted_iota(jnp.int32, (8, _LANES), 0).astype(jnp.float32)
    ridx = (jax.lax.broadcasted_iota(jnp.int32, (_SUPER, _LANES), 0) * _GROUP
            + jax.lax.broadcasted_iota(jnp.int32, (_SUPER, _LANES), 1))
    offs = list(range(8, n_cols - 8, 8)) + [n_cols - 8]

    for sg in range(_BLK // (_GROUP * _SUPER)):
        confs = []
        accs = []
        for g in range(_SUPER):
            c0 = (sg * _SUPER + g) * _GROUP
            e = jnp.exp(x_ref[:, c0:c0 + _GROUP])
            m = e[0:8, :]
            idx = jnp.zeros((8, _LANES), jnp.float32)
            for off in offs:
                ek = e[off:off + 8, :]
                cond = ek > m
                m = jnp.maximum(m, ek)
                idx = jnp.where(cond, jnp.float32(off), idx)
            cls = idx + sub8
            for sh in (4, 2, 1):
                mr = pltpu.roll(m, sh, 0)
                cr = pltpu.roll(cls, sh, 0)
                cond = mr > m
                m = jnp.maximum(m, mr)
                cls = jnp.where(cond, cr, cls)
            mx = m[0:1, :]
            pidx = cls[0:1, :]
            s = jnp.sum(e, axis=0, keepdims=True)
            labf = lab_ref[0:1, c0:c0 + _GROUP].astype(jnp.float32)
            confs.append(mx / s)
            accs.append(jnp.where(pidx == labf, 1.0, 0.0))
        conf8 = jnp.concatenate(confs, axis=0)
        acc8 = jnp.concatenate(accs, axis=0)
        base = i * _BLK + sg * _GROUP * _SUPER
        valid = (base + ridx) < n_total
        conf8 = jnp.where(valid, conf8, 0.0)
        for k in range(_N_BINS):
            gt = conf8 > bounds_ref[k]
            cnt_ref[k] += jnp.where(gt, 1.0, 0.0)
            sc_ref[k] += jnp.where(gt, conf8, 0.0)
            sa_ref[k] += jnp.where(gt, acc8, 0.0)

    @pl.when(i == n_steps - 1)
    def _fin():
        nf = jnp.float32(n_total)
        zero = jnp.zeros((1, 1), jnp.float32)
        gc = [jnp.sum(cnt_ref[k], keepdims=True)[:1, :1] for k in range(_N_BINS)]
        gs = [jnp.sum(sc_ref[k], keepdims=True)[:1, :1] for k in range(_N_BINS)]
        ga = [jnp.sum(sa_ref[k], keepdims=True)[:1, :1] for k in range(_N_BINS)]
        gc.append(zero)
        gs.append(zero)
        ga.append(zero)
        ece = zero
        oe = zero
        for b in range(_N_BINS):
            cnt = gc[b] - gc[b + 1]
            sc = gs[b] - gs[b + 1]
            sa = ga[b] - ga[b + 1]
            prop = cnt / nf
            denom = jnp.maximum(cnt, 1.0)
            accb = sa / denom
            avgc = sc / denom
            ce = avgc - accb
            ece = ece + jnp.abs(ce) * prop
            oe = oe + avgc * jnp.maximum(ce, 0.0) * prop
            accb_ref[pl.ds(b, 1), :] = accb
            prop_ref[pl.ds(b, 1), :] = prop
            ce_ref[pl.ds(b, 1), :] = jnp.abs(ce)
        ece_ref[...] = ece
        oe_ref[...] = oe


def kernel(logits, labels):
    n, c = logits.shape
    steps = pl.cdiv(n, _BLK)
    bounds = jnp.linspace(0.0, 1.0, _N_BINS + 1)
    lt = logits.T
    lab2 = labels.reshape(1, n)

    outs = pl.pallas_call(
        functools.partial(_ece_kernel, n_total=n, n_cols=c, n_steps=steps),
        grid=(steps,),
        in_specs=[
            pl.BlockSpec(memory_space=pltpu.SMEM),
            pl.BlockSpec((c, _BLK), lambda i: (0, i)),
            pl.BlockSpec((1, _BLK), lambda i: (0, i)),
        ],
        out_specs=[
            pl.BlockSpec((1, 1), lambda i: (0, 0)),
            pl.BlockSpec((_N_BINS, 1), lambda i: (0, 0)),
            pl.BlockSpec((1, 1), lambda i: (0, 0)),
            pl.BlockSpec((_N_BINS, 1), lambda i: (0, 0)),
            pl.BlockSpec((_N_BINS, 1), lambda i: (0, 0)),
        ],
        out_shape=[
            jax.ShapeDtypeStruct((1, 1), jnp.float32),
            jax.ShapeDtypeStruct((_N_BINS, 1), jnp.float32),
            jax.ShapeDtypeStruct((1, 1), jnp.float32),
            jax.ShapeDtypeStruct((_N_BINS, 1), jnp.float32),
            jax.ShapeDtypeStruct((_N_BINS, 1), jnp.float32),
        ],
        scratch_shapes=[pltpu.VMEM((_N_BINS, 8, _LANES), jnp.float32)] * 3,
        compiler_params=pltpu.CompilerParams(dimension_semantics=("arbitrary",)),
    )(bounds, lt, lab2)
    ece, accb, oe, prop, ce = outs
    return (ece.reshape(()), accb[:, 0], oe.reshape(()), prop[:, 0], ce[:, 0])


# speedup vs baseline: 9.6152x; 1.1807x over previous
import functools

import jax
import jax.numpy as jnp
from jax.experimental import pallas as pl
from jax.experimental.pallas import tpu as pltpu

_N_BINS = 10
_LANES = 128
_GROUP = 128
_SUPER = 8
_BLK = 32768


def _ece_kernel(bounds_ref, x_ref, lab_ref,
                ece_ref, accb_ref, oe_ref, prop_ref, ce_ref,
                cnt_ref, sc_ref, sa_ref,
                *, n_total, n_cols, n_steps):
    i = pl.program_id(0)

    @pl.when(i == 0)
    def _init():
        z = jnp.zeros((_N_BINS, 8, _LANES), jnp.float32)
        cnt_ref[...] = z
        sc_ref[...] = z
        sa_ref[...] = z

    sub8 = jax.lax.broadcasted_iota(jnp.int32, (8, _LANES), 0).astype(jnp.float32)
    ridx = (jax.lax.broadcasted_iota(jnp.int32, (_SUPER, _LANES), 0) * _GROUP
            + jax.lax.broadcasted_iota(jnp.int32, (_SUPER, _LANES), 1))
    offs = list(range(8, n_cols - 8, 8)) + [n_cols - 8]

    for sg in range(_BLK // (_GROUP * _SUPER)):
        confs = []
        accs = []
        for g in range(_SUPER):
            c0 = (sg * _SUPER + g) * _GROUP
            m = jnp.exp(x_ref[0:8, c0:c0 + _GROUP])
            s = m
            idx = jnp.zeros((8, _LANES), jnp.float32)
            for off in offs[:-1]:
                ek = jnp.exp(x_ref[off:off + 8, c0:c0 + _GROUP])
                cond = ek > m
                m = jnp.maximum(m, ek)
                idx = jnp.where(cond, jnp.float32(off), idx)
                s = s + ek
            off = offs[-1]
            ek = jnp.exp(x_ref[off:off + 8, c0:c0 + _GROUP])
            cond = ek > m
            m = jnp.maximum(m, ek)
            idx = jnp.where(cond, jnp.float32(off), idx)
            s = s + jnp.where(sub8 >= jnp.float32(offs[-2] + 8 - off),
                              ek, 0.0)
            cls = idx + sub8
            for sh in (4, 2, 1):
                mr = pltpu.roll(m, sh, 0)
                cr = pltpu.roll(cls, sh, 0)
                cond = mr > m
                m = jnp.maximum(m, mr)
                cls = jnp.where(cond, cr, cls)
                sr = pltpu.roll(s, sh, 0)
                s = s + sr
            mx = m[0:1, :]
            pidx = cls[0:1, :]
            ssum = s[0:1, :]
            labf = lab_ref[pl.ds(c0, _GROUP)].reshape(1, _GROUP).astype(jnp.float32)
            confs.append(mx / ssum)
            accs.append(jnp.where(pidx == labf, 1.0, 0.0))
        conf8 = jnp.concatenate(confs, axis=0)
        acc8 = jnp.concatenate(accs, axis=0)
        base = i * _BLK + sg * _GROUP * _SUPER
        valid = (base + ridx) < n_total
        conf8 = jnp.where(valid, conf8, 0.0)
        for k in range(_N_BINS):
            gt = conf8 > bounds_ref[k]
            cnt_ref[k] += jnp.where(gt, 1.0, 0.0)
            sc_ref[k] += jnp.where(gt, conf8, 0.0)
            sa_ref[k] += jnp.where(gt, acc8, 0.0)

    @pl.when(i == n_steps - 1)
    def _fin():
        nf = jnp.float32(n_total)
        zero = jnp.zeros((1, 1), jnp.float32)
        gc = [jnp.sum(cnt_ref[k], keepdims=True)[:1, :1] for k in range(_N_BINS)]
        gs = [jnp.sum(sc_ref[k], keepdims=True)[:1, :1] for k in range(_N_BINS)]
        ga = [jnp.sum(sa_ref[k], keepdims=True)[:1, :1] for k in range(_N_BINS)]
        gc.append(zero)
        gs.append(zero)
        ga.append(zero)
        ece = zero
        oe = zero
        for b in range(_N_BINS):
            cnt = gc[b] - gc[b + 1]
            sc = gs[b] - gs[b + 1]
            sa = ga[b] - ga[b + 1]
            prop = cnt / nf
            denom = jnp.maximum(cnt, 1.0)
            accb = sa / denom
            avgc = sc / denom
            ce = avgc - accb
            ece = ece + jnp.abs(ce) * prop
            oe = oe + avgc * jnp.maximum(ce, 0.0) * prop
            accb_ref[pl.ds(b, 1), :] = accb
            prop_ref[pl.ds(b, 1), :] = prop
            ce_ref[pl.ds(b, 1), :] = jnp.abs(ce)
        ece_ref[...] = ece
        oe_ref[...] = oe


def kernel(logits, labels):
    n, c = logits.shape
    steps = pl.cdiv(n, _BLK)
    bounds = jnp.linspace(0.0, 1.0, _N_BINS + 1)
    lt = logits.T

    outs = pl.pallas_call(
        functools.partial(_ece_kernel, n_total=n, n_cols=c, n_steps=steps),
        grid=(steps,),
        in_specs=[
            pl.BlockSpec(memory_space=pltpu.SMEM),
            pl.BlockSpec((c, _BLK), lambda i: (0, i)),
            pl.BlockSpec((_BLK,), lambda i: (i,)),
        ],
        out_specs=[
            pl.BlockSpec((1, 1), lambda i: (0, 0)),
            pl.BlockSpec((_N_BINS, 1), lambda i: (0, 0)),
            pl.BlockSpec((1, 1), lambda i: (0, 0)),
            pl.BlockSpec((_N_BINS, 1), lambda i: (0, 0)),
            pl.BlockSpec((_N_BINS, 1), lambda i: (0, 0)),
        ],
        out_shape=[
            jax.ShapeDtypeStruct((1, 1), jnp.float32),
            jax.ShapeDtypeStruct((_N_BINS, 1), jnp.float32),
            jax.ShapeDtypeStruct((1, 1), jnp.float32),
            jax.ShapeDtypeStruct((_N_BINS, 1), jnp.float32),
            jax.ShapeDtypeStruct((_N_BINS, 1), jnp.float32),
        ],
        scratch_shapes=[pltpu.VMEM((_N_BINS, 8, _LANES), jnp.float32)] * 3,
        compiler_params=pltpu.CompilerParams(dimension_semantics=("arbitrary",)),
    )(bounds, lt, labels)
    ece, accb, oe, prop, ce = outs
    return (ece.reshape(()), accb[:, 0], oe.reshape(()), prop[:, 0], ce[:, 0])


# speedup vs baseline: 9.7312x; 1.0121x over previous
import functools

import jax
import jax.numpy as jnp
from jax.experimental import pallas as pl
from jax.experimental.pallas import tpu as pltpu

_N_BINS = 10
_LANES = 128
_GROUP = 128
_SUPER = 8
_BLK = 32768


def _ece_kernel(bounds_ref, x_ref, lab_ref,
                ece_ref, accb_ref, oe_ref, prop_ref, ce_ref,
                cnt_ref, sc_ref, sa_ref, confrow_ref, accrow_ref,
                *, n_total, n_cols, n_steps):
    i = pl.program_id(0)

    @pl.when(i == 0)
    def _init():
        z = jnp.zeros((_N_BINS, 8, _LANES), jnp.float32)
        cnt_ref[...] = z
        sc_ref[...] = z
        sa_ref[...] = z

    sub8 = jax.lax.broadcasted_iota(jnp.int32, (8, _LANES), 0).astype(jnp.float32)
    ridx = (jax.lax.broadcasted_iota(jnp.int32, (_SUPER, _LANES), 0) * _GROUP
            + jax.lax.broadcasted_iota(jnp.int32, (_SUPER, _LANES), 1))
    offs = list(range(8, n_cols - 8, 8)) + [n_cols - 8]

    for sg in range(_BLK // (_GROUP * _SUPER)):
        for g in range(_SUPER):
            c0 = (sg * _SUPER + g) * _GROUP
            m = jnp.exp(x_ref[0:8, c0:c0 + _GROUP])
            s = m
            idx = jnp.zeros((8, _LANES), jnp.float32)
            for off in offs[:-1]:
                ek = jnp.exp(x_ref[off:off + 8, c0:c0 + _GROUP])
                cond = ek > m
                m = jnp.maximum(m, ek)
                idx = jnp.where(cond, jnp.float32(off), idx)
                s = s + ek
            off = offs[-1]
            ek = jnp.exp(x_ref[off:off + 8, c0:c0 + _GROUP])
            cond = ek > m
            m = jnp.maximum(m, ek)
            idx = jnp.where(cond, jnp.float32(off), idx)
            s = s + jnp.where(sub8 >= jnp.float32(offs[-2] + 8 - off),
                              ek, 0.0)
            cls = idx + sub8
            for sh in (4, 2, 1):
                mr = pltpu.roll(m, sh, 0)
                cr = pltpu.roll(cls, sh, 0)
                cond = mr > m
                m = jnp.maximum(m, mr)
                cls = jnp.where(cond, cr, cls)
                sr = pltpu.roll(s, sh, 0)
                s = s + sr
            mx = m[0:1, :]
            pidx = cls[0:1, :]
            ssum = s[0:1, :]
            labf = lab_ref[pl.ds(c0, _GROUP)].reshape(1, _GROUP).astype(jnp.float32)
            confrow_ref[pl.ds(g, 1), :] = mx / ssum
            accrow_ref[pl.ds(g, 1), :] = jnp.where(pidx == labf, 1.0, 0.0)
        conf8 = confrow_ref[...]
        acc8 = accrow_ref[...]
        base = i * _BLK + sg * _GROUP * _SUPER
        valid = (base + ridx) < n_total
        conf8 = jnp.where(valid, conf8, 0.0)
        for k in range(_N_BINS):
            gt = conf8 > bounds_ref[k]
            cnt_ref[k] += jnp.where(gt, 1.0, 0.0)
            sc_ref[k] += jnp.where(gt, conf8, 0.0)
            sa_ref[k] += jnp.where(gt, acc8, 0.0)

    @pl.when(i == n_steps - 1)
    def _fin():
        nf = jnp.float32(n_total)
        zero = jnp.zeros((1, 1), jnp.float32)
        gc = [jnp.sum(cnt_ref[k], keepdims=True)[:1, :1] for k in range(_N_BINS)]
        gs = [jnp.sum(sc_ref[k], keepdims=True)[:1, :1] for k in range(_N_BINS)]
        ga = [jnp.sum(sa_ref[k], keepdims=True)[:1, :1] for k in range(_N_BINS)]
        gc.append(zero)
        gs.append(zero)
        ga.append(zero)
        ece = zero
        oe = zero
        for b in range(_N_BINS):
            cnt = gc[b] - gc[b + 1]
            sc = gs[b] - gs[b + 1]
            sa = ga[b] - ga[b + 1]
            prop = cnt / nf
            denom = jnp.maximum(cnt, 1.0)
            accb = sa / denom
            avgc = sc / denom
            ce = avgc - accb
            ece = ece + jnp.abs(ce) * prop
            oe = oe + avgc * jnp.maximum(ce, 0.0) * prop
            accb_ref[pl.ds(b, 1), :] = accb
            prop_ref[pl.ds(b, 1), :] = prop
            ce_ref[pl.ds(b, 1), :] = jnp.abs(ce)
        ece_ref[...] = ece
        oe_ref[...] = oe


def kernel(logits, labels):
    n, c = logits.shape
    steps = pl.cdiv(n, _BLK)
    bounds = jnp.linspace(0.0, 1.0, _N_BINS + 1)
    lt = logits.T

    outs = pl.pallas_call(
        functools.partial(_ece_kernel, n_total=n, n_cols=c, n_steps=steps),
        grid=(steps,),
        in_specs=[
            pl.BlockSpec(memory_space=pltpu.SMEM),
            pl.BlockSpec((c, _BLK), lambda i: (0, i)),
            pl.BlockSpec((_BLK,), lambda i: (i,)),
        ],
        out_specs=[
            pl.BlockSpec((1, 1), lambda i: (0, 0)),
            pl.BlockSpec((_N_BINS, 1), lambda i: (0, 0)),
            pl.BlockSpec((1, 1), lambda i: (0, 0)),
            pl.BlockSpec((_N_BINS, 1), lambda i: (0, 0)),
            pl.BlockSpec((_N_BINS, 1), lambda i: (0, 0)),
        ],
        out_shape=[
            jax.ShapeDtypeStruct((1, 1), jnp.float32),
            jax.ShapeDtypeStruct((_N_BINS, 1), jnp.float32),
            jax.ShapeDtypeStruct((1, 1), jnp.float32),
            jax.ShapeDtypeStruct((_N_BINS, 1), jnp.float32),
            jax.ShapeDtypeStruct((_N_BINS, 1), jnp.float32),
        ],
        scratch_shapes=[pltpu.VMEM((_N_BINS, 8, _LANES), jnp.float32)] * 3
        + [pltpu.VMEM((8, _LANES), jnp.float32)] * 2,
        compiler_params=pltpu.CompilerParams(dimension_semantics=("arbitrary",)),
    )(bounds, lt, labels)
    ece, accb, oe, prop, ce = outs
    return (ece.reshape(()), accb[:, 0], oe.reshape(()), prop[:, 0], ce[:, 0])


# speedup vs baseline: 9.7423x; 1.0011x over previous
import functools

import jax
import jax.numpy as jnp
from jax.experimental import pallas as pl
from jax.experimental.pallas import tpu as pltpu

_N_BINS = 10
_LANES = 128
_GROUP = 128
_SUPER = 8
_BLK = 49152


def _ece_kernel(bounds_ref, x_ref, lab_ref,
                ece_ref, accb_ref, oe_ref, prop_ref, ce_ref,
                cnt_ref, sc_ref, sa_ref, confrow_ref, accrow_ref,
                *, n_total, n_cols, n_steps):
    i = pl.program_id(0)

    @pl.when(i == 0)
    def _init():
        z = jnp.zeros((_N_BINS, 8, _LANES), jnp.float32)
        cnt_ref[...] = z
        sc_ref[...] = z
        sa_ref[...] = z

    sub8 = jax.lax.broadcasted_iota(jnp.int32, (8, _LANES), 0).astype(jnp.float32)
    ridx = (jax.lax.broadcasted_iota(jnp.int32, (_SUPER, _LANES), 0) * _GROUP
            + jax.lax.broadcasted_iota(jnp.int32, (_SUPER, _LANES), 1))
    offs = list(range(8, n_cols - 8, 8)) + [n_cols - 8]

    for sg in range(_BLK // (_GROUP * _SUPER)):
        for g in range(_SUPER):
            c0 = (sg * _SUPER + g) * _GROUP
            m = jnp.exp(x_ref[0:8, c0:c0 + _GROUP])
            s = m
            idx = jnp.zeros((8, _LANES), jnp.float32)
            for off in offs[:-1]:
                ek = jnp.exp(x_ref[off:off + 8, c0:c0 + _GROUP])
                cond = ek > m
                m = jnp.maximum(m, ek)
                idx = jnp.where(cond, jnp.float32(off), idx)
                s = s + ek
            off = offs[-1]
            ek = jnp.exp(x_ref[off:off + 8, c0:c0 + _GROUP])
            cond = ek > m
            m = jnp.maximum(m, ek)
            idx = jnp.where(cond, jnp.float32(off), idx)
            s = s + jnp.where(sub8 >= jnp.float32(offs[-2] + 8 - off),
                              ek, 0.0)
            cls = idx + sub8
            for sh in (4, 2, 1):
                mr = pltpu.roll(m, sh, 0)
                cr = pltpu.roll(cls, sh, 0)
                cond = mr > m
                m = jnp.maximum(m, mr)
                cls = jnp.where(cond, cr, cls)
                sr = pltpu.roll(s, sh, 0)
                s = s + sr
            mx = m[0:1, :]
            pidx = cls[0:1, :]
            ssum = s[0:1, :]
            labf = lab_ref[pl.ds(c0, _GROUP)].reshape(1, _GROUP).astype(jnp.float32)
            confrow_ref[pl.ds(g, 1), :] = mx / ssum
            accrow_ref[pl.ds(g, 1), :] = jnp.where(pidx == labf, 1.0, 0.0)
        conf8 = confrow_ref[...]
        acc8 = accrow_ref[...]
        base = i * _BLK + sg * _GROUP * _SUPER
        valid = (base + ridx) < n_total
        conf8 = jnp.where(valid, conf8, 0.0)
        for k in range(_N_BINS):
            gt = conf8 > bounds_ref[k]
            cnt_ref[k] += jnp.where(gt, 1.0, 0.0)
            sc_ref[k] += jnp.where(gt, conf8, 0.0)
            sa_ref[k] += jnp.where(gt, acc8, 0.0)

    @pl.when(i == n_steps - 1)
    def _fin():
        nf = jnp.float32(n_total)
        zero = jnp.zeros((1, 1), jnp.float32)
        gc = [jnp.sum(cnt_ref[k], keepdims=True)[:1, :1] for k in range(_N_BINS)]
        gs = [jnp.sum(sc_ref[k], keepdims=True)[:1, :1] for k in range(_N_BINS)]
        ga = [jnp.sum(sa_ref[k], keepdims=True)[:1, :1] for k in range(_N_BINS)]
        gc.append(zero)
        gs.append(zero)
        ga.append(zero)
        ece = zero
        oe = zero
        for b in range(_N_BINS):
            cnt = gc[b] - gc[b + 1]
            sc = gs[b] - gs[b + 1]
            sa = ga[b] - ga[b + 1]
            prop = cnt / nf
            denom = jnp.maximum(cnt, 1.0)
            accb = sa / denom
            avgc = sc / denom
            ce = avgc - accb
            ece = ece + jnp.abs(ce) * prop
            oe = oe + avgc * jnp.maximum(ce, 0.0) * prop
            accb_ref[pl.ds(b, 1), :] = accb
            prop_ref[pl.ds(b, 1), :] = prop
            ce_ref[pl.ds(b, 1), :] = jnp.abs(ce)
        ece_ref[...] = ece
        oe_ref[...] = oe


def kernel(logits, labels):
    n, c = logits.shape
    steps = pl.cdiv(n, _BLK)
    bounds = jnp.linspace(0.0, 1.0, _N_BINS + 1)
    lt = logits.T

    outs = pl.pallas_call(
        functools.partial(_ece_kernel, n_total=n, n_cols=c, n_steps=steps),
        grid=(steps,),
        in_specs=[
            pl.BlockSpec(memory_space=pltpu.SMEM),
            pl.BlockSpec((c, _BLK), lambda i: (0, i)),
            pl.BlockSpec((_BLK,), lambda i: (i,)),
        ],
        out_specs=[
            pl.BlockSpec((1, 1), lambda i: (0, 0)),
            pl.BlockSpec((_N_BINS, 1), lambda i: (0, 0)),
            pl.BlockSpec((1, 1), lambda i: (0, 0)),
            pl.BlockSpec((_N_BINS, 1), lambda i: (0, 0)),
            pl.BlockSpec((_N_BINS, 1), lambda i: (0, 0)),
        ],
        out_shape=[
            jax.ShapeDtypeStruct((1, 1), jnp.float32),
            jax.ShapeDtypeStruct((_N_BINS, 1), jnp.float32),
            jax.ShapeDtypeStruct((1, 1), jnp.float32),
            jax.ShapeDtypeStruct((_N_BINS, 1), jnp.float32),
            jax.ShapeDtypeStruct((_N_BINS, 1), jnp.float32),
        ],
        scratch_shapes=[pltpu.VMEM((_N_BINS, 8, _LANES), jnp.float32)] * 3
        + [pltpu.VMEM((8, _LANES), jnp.float32)] * 2,
        compiler_params=pltpu.CompilerParams(dimension_semantics=("arbitrary",)),
    )(bounds, lt, labels)
    ece, accb, oe, prop, ce = outs
    return (ece.reshape(()), accb[:, 0], oe.reshape(()), prop[:, 0], ce[:, 0])
